# baseline (speedup 1.0000x reference)
import dataclasses
import functools

import jax
import jax.numpy as jnp
from jax import lax
from jax.experimental import pallas as pl
from jax.experimental.pallas import tpu as pltpu
from jax.experimental.pallas import tpu_sc as plsc

E = 64
H = 768
I = 1536
N = 2048
T = 64
NT = 96
NPAD = NT * T

NW = 32
CH = N // NW
LANES = 16


def _cumsum_lanes(m):
    col = lax.broadcasted_iota(jnp.int32, m.shape, 1)
    sh = 1
    while sh < m.shape[1]:
        m = m + jnp.where(col >= sh, pltpu.roll(m, sh, axis=1), 0)
        sh *= 2
    return m


def _routing_body(idx_ref, pos_ref, eot_ref):
    idx = idx_ref[...]
    e_iota = lax.broadcasted_iota(jnp.int32, (E, N), 0)
    m = (idx == e_iota).astype(jnp.int32)
    csum = _cumsum_lanes(m)
    rank = csum - m
    counts = csum[:, N - 1:N]
    ntiles = (counts + (T - 1)) // T
    tri = (lax.broadcasted_iota(jnp.int32, (E, E), 0)
           > lax.broadcasted_iota(jnp.int32, (E, E), 1)).astype(jnp.float32)
    tile_start = jnp.dot(tri, ntiles.astype(jnp.float32),
                         preferred_element_type=jnp.float32).astype(jnp.int32)
    pad_off = tile_start * T
    pos = jnp.sum(m * (rank + pad_off), axis=0, keepdims=True)
    pos_ref[...] = pos

    i_iota = lax.broadcasted_iota(jnp.int32, (E, NT), 1)
    act = (i_iota >= tile_start) & (i_iota < tile_start + ntiles)
    e_iota2 = lax.broadcasted_iota(jnp.int32, (E, NT), 0)
    eot = jnp.sum(jnp.where(act, e_iota2, 0), axis=0, keepdims=True)
    r_total = jnp.sum(ntiles)
    eot_ref[...] = jnp.where(
        lax.broadcasted_iota(jnp.int32, (1, NT), 1) >= r_total, E - 1, eot)


def _routing(idx2d):
    return pl.pallas_call(
        _routing_body,
        out_shape=[jax.ShapeDtypeStruct((1, N), jnp.int32),
                   jax.ShapeDtypeStruct((1, NT), jnp.int32)],
    )(idx2d)


@functools.lru_cache(maxsize=None)
def _sc_kernels():
    mesh = plsc.VectorSubcoreMesh(core_axis_name="c", subcore_axis_name="s")

    @functools.partial(
        pl.kernel, mesh=mesh,
        out_type=jax.ShapeDtypeStruct((NPAD, H), jnp.float32),
        scratch_types=[pltpu.VMEM((CH,), jnp.int32),
                       pltpu.VMEM((CH, H), jnp.float32),
                       pltpu.SemaphoreType.DMA])
    def sc_scatter(x_hbm, pos_hbm, xs_hbm, idx_v, rows_v, sem):
        wid = lax.axis_index("s") * 2 + lax.axis_index("c")
        base = wid * CH
        pltpu.sync_copy(pos_hbm.at[pl.ds(base, CH)], idx_v)
        pltpu.sync_copy(x_hbm.at[pl.ds(base, CH)], rows_v)
        pltpu.async_copy(rows_v, xs_hbm.at[idx_v], sem).wait()

    cp = pltpu.CompilerParams()
    if "needs_layout_passes" in pltpu.CompilerParams.__dataclass_fields__:
        cp = dataclasses.replace(cp, needs_layout_passes=False)

    @functools.partial(
        pl.kernel, mesh=mesh,
        out_type=jax.ShapeDtypeStruct((N, H), jnp.float32),
        compiler_params=cp,
        scratch_types=[pltpu.VMEM((CH,), jnp.int32),
                       pltpu.VMEM((CH,), jnp.float32),
                       pltpu.VMEM((CH, H), jnp.float32),
                       pltpu.SemaphoreType.DMA])
    def sc_gather(ys_hbm, pos_hbm, ew_hbm, out_hbm, idx_v, w_v, rows_v, sem):
        wid = lax.axis_index("s") * 2 + lax.axis_index("c")
        base = wid * CH
        pltpu.sync_copy(pos_hbm.at[pl.ds(base, CH)], idx_v)
        pltpu.sync_copy(ew_hbm.at[pl.ds(base, CH)], w_v)
        pltpu.async_copy(ys_hbm.at[idx_v], rows_v, sem).wait()

        @pl.loop(0, CH)
        def _row(i):
            wb = plsc.load_gather(w_v, [jnp.full((LANES,), 0, jnp.int32) + i])

            @pl.loop(0, H, step=LANES)
            def _col(j):
                rows_v[i, pl.ds(j, LANES)] = rows_v[i, pl.ds(j, LANES)] * wb

        pltpu.sync_copy(rows_v, out_hbm.at[pl.ds(base, CH)])

    return sc_scatter, sc_gather


def _gemm_body(eot_ref, x_ref, g_ref, u_ref, d_ref, y_ref, gb, ub, db):
    i = pl.program_id(0)
    changed = jnp.logical_or(
        i == 0, eot_ref[i] != eot_ref[jnp.maximum(i - 1, 0)])

    @pl.when(changed)
    def _():
        gb[...] = g_ref[0].astype(jnp.bfloat16)
        ub[...] = u_ref[0].astype(jnp.bfloat16)
        db[...] = d_ref[0].astype(jnp.bfloat16)

    xt = x_ref[...].astype(jnp.bfloat16)
    g = lax.dot_general(xt, gb[...], (((1,), (1,)), ((), ())),
                        preferred_element_type=jnp.float32)
    u = lax.dot_general(xt, ub[...], (((1,), (1,)), ((), ())),
                        preferred_element_type=jnp.float32)
    inter = (g * (1.0 / (1.0 + jnp.exp(-g))) * u).astype(jnp.bfloat16)
    y_ref[...] = lax.dot_general(inter, db[...], (((1,), (1,)), ((), ())),
                                 preferred_element_type=jnp.float32)


def _gemm(eot, xs, gate_proj, up_proj, down_proj):
    grid_spec = pltpu.PrefetchScalarGridSpec(
        num_scalar_prefetch=1,
        grid=(NT,),
        in_specs=[
            pl.BlockSpec((T, H), lambda i, eot: (i, 0)),
            pl.BlockSpec((1, I, H), lambda i, eot: (eot[i], 0, 0)),
            pl.BlockSpec((1, I, H), lambda i, eot: (eot[i], 0, 0)),
            pl.BlockSpec((1, H, I), lambda i, eot: (eot[i], 0, 0)),
        ],
        out_specs=pl.BlockSpec((T, H), lambda i, eot: (i, 0)),
        scratch_shapes=[pltpu.VMEM((I, H), jnp.bfloat16),
                        pltpu.VMEM((I, H), jnp.bfloat16),
                        pltpu.VMEM((H, I), jnp.bfloat16)],
    )
    return pl.pallas_call(
        _gemm_body,
        grid_spec=grid_spec,
        out_shape=jax.ShapeDtypeStruct((NPAD, H), jnp.float32),
    )(eot, xs, gate_proj, up_proj, down_proj)


def kernel(x, expert_indices, expert_weights, gate_proj, up_proj, down_proj):
    b, s, h = x.shape
    x2 = x.reshape(s, h)
    idx2 = expert_indices.reshape(1, s).astype(jnp.int32)
    ew = expert_weights.reshape(s)

    pos2, eot2 = _routing(idx2)
    pos = pos2.reshape(s)
    eot = eot2.reshape(NT)

    sc_scatter, sc_gather = _sc_kernels()
    xs = sc_scatter(x2, pos)
    ys = _gemm(eot, xs, gate_proj, up_proj, down_proj)
    out = sc_gather(ys, pos, ew)
    return out.reshape(b, s, h)


# speedup vs baseline: 1.0314x; 1.0314x over previous
import dataclasses
import functools

import jax
import jax.numpy as jnp
from jax import lax
from jax.experimental import pallas as pl
from jax.experimental.pallas import tpu as pltpu
from jax.experimental.pallas import tpu_sc as plsc

E = 64
H = 768
I = 1536
N = 2048
T = 64
NT = 96
NPAD = NT * T

NW = 32
CH = N // NW
LANES = 16
WREP = 128


def _cumsum_lanes(m):
    col = lax.broadcasted_iota(jnp.int32, m.shape, 1)
    sh = 1
    while sh < m.shape[1]:
        m = m + jnp.where(col >= sh, pltpu.roll(m, sh, axis=1), 0)
        sh *= 2
    return m


def _routing_body(idx_ref, pos_ref, eot_ref):
    idx = idx_ref[...]
    e_iota = lax.broadcasted_iota(jnp.int32, (E, N), 0)
    m = (idx == e_iota).astype(jnp.int32)
    csum = _cumsum_lanes(m)
    rank = csum - m
    counts = csum[:, N - 1:N]
    ntiles = (counts + (T - 1)) // T
    tri = (lax.broadcasted_iota(jnp.int32, (E, E), 0)
           > lax.broadcasted_iota(jnp.int32, (E, E), 1)).astype(jnp.float32)
    tile_start = jnp.dot(tri, ntiles.astype(jnp.float32),
                         preferred_element_type=jnp.float32).astype(jnp.int32)
    pad_off = tile_start * T
    pos = jnp.sum(m * (rank + pad_off), axis=0, keepdims=True)
    pos_ref[...] = pos

    i_iota = lax.broadcasted_iota(jnp.int32, (E, NT), 1)
    act = (i_iota >= tile_start) & (i_iota < tile_start + ntiles)
    e_iota2 = lax.broadcasted_iota(jnp.int32, (E, NT), 0)
    eot = jnp.sum(jnp.where(act, e_iota2, 0), axis=0, keepdims=True)
    r_total = jnp.sum(ntiles)
    eot_ref[...] = jnp.where(
        lax.broadcasted_iota(jnp.int32, (1, NT), 1) >= r_total, E - 1, eot)


def _routing(idx2d):
    return pl.pallas_call(
        _routing_body,
        out_shape=[jax.ShapeDtypeStruct((1, N), jnp.int32),
                   jax.ShapeDtypeStruct((1, NT), jnp.int32)],
    )(idx2d)


@functools.lru_cache(maxsize=None)
def _sc_kernels():
    mesh = plsc.VectorSubcoreMesh(core_axis_name="c", subcore_axis_name="s")
    cp = pltpu.CompilerParams()
    if "needs_layout_passes" in pltpu.CompilerParams.__dataclass_fields__:
        cp = dataclasses.replace(cp, needs_layout_passes=False)

    @functools.partial(
        pl.kernel, mesh=mesh,
        out_type=[jax.ShapeDtypeStruct((NPAD, H), jnp.float32),
                  jax.ShapeDtypeStruct((NPAD, WREP), jnp.float32)],
        compiler_params=cp,
        scratch_types=[pltpu.VMEM((CH,), jnp.int32),
                       pltpu.VMEM((CH,), jnp.float32),
                       pltpu.VMEM((CH, H), jnp.float32),
                       pltpu.VMEM((CH, WREP), jnp.float32),
                       pltpu.SemaphoreType.DMA,
                       pltpu.SemaphoreType.DMA])
    def sc_scatter(x_hbm, pos_hbm, ew_hbm, xs_hbm, ws_hbm,
                   idx_v, w_v, rows_v, wrep_v, sem, sem2):
        wid = lax.axis_index("s") * 2 + lax.axis_index("c")
        base = wid * CH
        pltpu.sync_copy(pos_hbm.at[pl.ds(base, CH)], idx_v)
        pltpu.sync_copy(ew_hbm.at[pl.ds(base, CH)], w_v)
        pltpu.sync_copy(x_hbm.at[pl.ds(base, CH)], rows_v)

        @pl.loop(0, CH)
        def _row(i):
            wb = plsc.load_gather(w_v, [jnp.full((LANES,), 0, jnp.int32) + i])

            @pl.loop(0, WREP, step=LANES)
            def _col(j):
                wrep_v[i, pl.ds(j, LANES)] = wb

        c1 = pltpu.async_copy(rows_v, xs_hbm.at[idx_v], sem)
        c2 = pltpu.async_copy(wrep_v, ws_hbm.at[idx_v], sem2)
        c1.wait()
        c2.wait()

    @functools.partial(
        pl.kernel, mesh=mesh,
        out_type=jax.ShapeDtypeStruct((N, H), jnp.float32),
        scratch_types=[pltpu.VMEM((CH,), jnp.int32),
                       pltpu.VMEM((CH, H), jnp.float32),
                       pltpu.SemaphoreType.DMA])
    def sc_gather(ys_hbm, pos_hbm, out_hbm, idx_v, rows_v, sem):
        wid = lax.axis_index("s") * 2 + lax.axis_index("c")
        base = wid * CH
        pltpu.sync_copy(pos_hbm.at[pl.ds(base, CH)], idx_v)
        pltpu.async_copy(ys_hbm.at[idx_v], rows_v, sem).wait()
        pltpu.sync_copy(rows_v, out_hbm.at[pl.ds(base, CH)])

    return sc_scatter, sc_gather


def _gemm_body(eot_ref, x_ref, g_ref, u_ref, d_ref, w_ref, y_ref, gb, ub, db):
    i = pl.program_id(0)
    changed = jnp.logical_or(
        i == 0, eot_ref[i] != eot_ref[jnp.maximum(i - 1, 0)])

    @pl.when(changed)
    def _():
        gb[...] = g_ref[0].astype(jnp.bfloat16)
        ub[...] = u_ref[0].astype(jnp.bfloat16)
        db[...] = d_ref[0].astype(jnp.bfloat16)

    xt = x_ref[...].astype(jnp.bfloat16)
    g = lax.dot_general(xt, gb[...], (((1,), (1,)), ((), ())),
                        preferred_element_type=jnp.float32)
    u = lax.dot_general(xt, ub[...], (((1,), (1,)), ((), ())),
                        preferred_element_type=jnp.float32)
    inter = (g * (1.0 / (1.0 + jnp.exp(-g))) * u).astype(jnp.bfloat16)
    y = lax.dot_general(inter, db[...], (((1,), (1,)), ((), ())),
                        preferred_element_type=jnp.float32)
    y_ref[...] = y * w_ref[:, 0:1]


def _gemm(eot, xs, gate_proj, up_proj, down_proj, ws):
    grid_spec = pltpu.PrefetchScalarGridSpec(
        num_scalar_prefetch=1,
        grid=(NT,),
        in_specs=[
            pl.BlockSpec((T, H), lambda i, eot: (i, 0)),
            pl.BlockSpec((1, I, H), lambda i, eot: (eot[i], 0, 0)),
            pl.BlockSpec((1, I, H), lambda i, eot: (eot[i], 0, 0)),
            pl.BlockSpec((1, H, I), lambda i, eot: (eot[i], 0, 0)),
            pl.BlockSpec((T, WREP), lambda i, eot: (i, 0)),
        ],
        out_specs=pl.BlockSpec((T, H), lambda i, eot: (i, 0)),
        scratch_shapes=[pltpu.VMEM((I, H), jnp.bfloat16),
                        pltpu.VMEM((I, H), jnp.bfloat16),
                        pltpu.VMEM((H, I), jnp.bfloat16)],
    )
    return pl.pallas_call(
        _gemm_body,
        grid_spec=grid_spec,
        out_shape=jax.ShapeDtypeStruct((NPAD, H), jnp.float32),
    )(eot, xs, gate_proj, up_proj, down_proj, ws)


def kernel(x, expert_indices, expert_weights, gate_proj, up_proj, down_proj):
    b, s, h = x.shape
    x2 = x.reshape(s, h)
    idx2 = expert_indices.reshape(1, s).astype(jnp.int32)
    ew = expert_weights.reshape(s)

    pos2, eot2 = _routing(idx2)
    pos = pos2.reshape(s)
    eot = eot2.reshape(NT)

    sc_scatter, sc_gather = _sc_kernels()
    xs, ws = sc_scatter(x2, pos, ew)
    ys = _gemm(eot, xs, gate_proj, up_proj, down_proj, ws)
    out = sc_gather(ys, pos)
    return out.reshape(b, s, h)


# speedup vs baseline: 1.0356x; 1.0040x over previous
import dataclasses
import functools

import jax
import jax.numpy as jnp
from jax import lax
from jax.experimental import pallas as pl
from jax.experimental.pallas import tpu as pltpu
from jax.experimental.pallas import tpu_sc as plsc

E = 64
H = 768
I = 1536
N = 2048
T = 64
NT = 96
NPAD = NT * T

NW = 32
CH = N // NW
LANES = 16
WREP = 128


def _cumsum_lanes(m):
    col = lax.broadcasted_iota(jnp.int32, m.shape, 1)
    sh = 1
    while sh < m.shape[1]:
        m = m + jnp.where(col >= sh, pltpu.roll(m, sh, axis=1), 0)
        sh *= 2
    return m


def _routing_body(idx_ref, pos_ref, eot_ref):
    idx = idx_ref[...]
    e_iota = lax.broadcasted_iota(jnp.int32, (E, N), 0)
    m = (idx == e_iota).astype(jnp.int32)
    csum = _cumsum_lanes(m)
    rank = csum - m
    counts = csum[:, N - 1:N]
    ntiles = (counts + (T - 1)) // T
    tri = (lax.broadcasted_iota(jnp.int32, (E, E), 0)
           > lax.broadcasted_iota(jnp.int32, (E, E), 1)).astype(jnp.float32)
    tile_start = jnp.dot(tri, ntiles.astype(jnp.float32),
                         preferred_element_type=jnp.float32).astype(jnp.int32)
    pad_off = tile_start * T
    pos = jnp.sum(m * (rank + pad_off), axis=0, keepdims=True)
    pos_ref[...] = pos

    i_iota = lax.broadcasted_iota(jnp.int32, (E, NT), 1)
    act = (i_iota >= tile_start) & (i_iota < tile_start + ntiles)
    e_iota2 = lax.broadcasted_iota(jnp.int32, (E, NT), 0)
    eot = jnp.sum(jnp.where(act, e_iota2, 0), axis=0, keepdims=True)
    r_total = jnp.sum(ntiles)
    eot_ref[...] = jnp.where(
        lax.broadcasted_iota(jnp.int32, (1, NT), 1) >= r_total, E - 1, eot)


def _routing(idx2d):
    return pl.pallas_call(
        _routing_body,
        out_shape=[jax.ShapeDtypeStruct((1, N), jnp.int32),
                   jax.ShapeDtypeStruct((1, NT), jnp.int32)],
    )(idx2d)


@functools.lru_cache(maxsize=None)
def _sc_kernels():
    mesh = plsc.VectorSubcoreMesh(core_axis_name="c", subcore_axis_name="s")
    cp = pltpu.CompilerParams()
    if "needs_layout_passes" in pltpu.CompilerParams.__dataclass_fields__:
        cp = dataclasses.replace(cp, needs_layout_passes=False)

    @functools.partial(
        pl.kernel, mesh=mesh,
        out_type=[jax.ShapeDtypeStruct((NPAD, H), jnp.float32),
                  jax.ShapeDtypeStruct((NPAD, WREP), jnp.float32)],
        compiler_params=cp,
        scratch_types=[pltpu.VMEM((CH,), jnp.int32),
                       pltpu.VMEM((CH,), jnp.float32),
                       pltpu.VMEM((CH, H), jnp.float32),
                       pltpu.VMEM((CH, WREP), jnp.float32),
                       pltpu.SemaphoreType.DMA,
                       pltpu.SemaphoreType.DMA])
    def sc_scatter(x_hbm, pos_hbm, ew_hbm, xs_hbm, ws_hbm,
                   idx_v, w_v, rows_v, wrep_v, sem, sem2):
        wid = lax.axis_index("s") * 2 + lax.axis_index("c")
        base = wid * CH
        pltpu.sync_copy(pos_hbm.at[pl.ds(base, CH)], idx_v)
        pltpu.sync_copy(ew_hbm.at[pl.ds(base, CH)], w_v)
        pltpu.sync_copy(x_hbm.at[pl.ds(base, CH)], rows_v)

        @pl.loop(0, CH)
        def _row(i):
            wb = plsc.load_gather(w_v, [jnp.full((LANES,), 0, jnp.int32) + i])

            @pl.loop(0, WREP, step=LANES)
            def _col(j):
                wrep_v[i, pl.ds(j, LANES)] = wb

        c1 = pltpu.async_copy(rows_v, xs_hbm.at[idx_v], sem)
        c2 = pltpu.async_copy(wrep_v, ws_hbm.at[idx_v], sem2)
        c1.wait()
        c2.wait()

    @functools.partial(
        pl.kernel, mesh=mesh,
        out_type=jax.ShapeDtypeStruct((N, H), jnp.float32),
        scratch_types=[pltpu.VMEM((CH,), jnp.int32),
                       pltpu.VMEM((CH, H), jnp.float32),
                       pltpu.SemaphoreType.DMA])
    def sc_gather(ys_hbm, pos_hbm, out_hbm, idx_v, rows_v, sem):
        wid = lax.axis_index("s") * 2 + lax.axis_index("c")
        base = wid * CH
        pltpu.sync_copy(pos_hbm.at[pl.ds(base, CH)], idx_v)
        pltpu.async_copy(ys_hbm.at[idx_v], rows_v, sem).wait()
        pltpu.sync_copy(rows_v, out_hbm.at[pl.ds(base, CH)])

    return sc_scatter, sc_gather


def _gemm_body(eot_ref, x_ref, g_ref, u_ref, d_ref, w_ref, y_ref, gub, db):
    i = pl.program_id(0)
    changed = jnp.logical_or(
        i == 0, eot_ref[i] != eot_ref[jnp.maximum(i - 1, 0)])

    @pl.when(changed)
    def _():
        gub[0:I, :] = g_ref[0].astype(jnp.bfloat16)
        gub[I:2 * I, :] = u_ref[0].astype(jnp.bfloat16)
        db[...] = d_ref[0].astype(jnp.bfloat16)

    xt = x_ref[...].astype(jnp.bfloat16)
    gu = lax.dot_general(xt, gub[...], (((1,), (1,)), ((), ())),
                         preferred_element_type=jnp.float32)
    g = gu[:, 0:I]
    u = gu[:, I:2 * I]
    inter = (g * (1.0 / (1.0 + jnp.exp(-g))) * u).astype(jnp.bfloat16)
    y = lax.dot_general(inter, db[...], (((1,), (1,)), ((), ())),
                        preferred_element_type=jnp.float32)
    y_ref[...] = y * w_ref[:, 0:1]


def _gemm(eot, xs, gate_proj, up_proj, down_proj, ws):
    grid_spec = pltpu.PrefetchScalarGridSpec(
        num_scalar_prefetch=1,
        grid=(NT,),
        in_specs=[
            pl.BlockSpec((T, H), lambda i, eot: (i, 0)),
            pl.BlockSpec((1, I, H), lambda i, eot: (eot[i], 0, 0)),
            pl.BlockSpec((1, I, H), lambda i, eot: (eot[i], 0, 0)),
            pl.BlockSpec((1, H, I), lambda i, eot: (eot[i], 0, 0)),
            pl.BlockSpec((T, WREP), lambda i, eot: (i, 0)),
        ],
        out_specs=pl.BlockSpec((T, H), lambda i, eot: (i, 0)),
        scratch_shapes=[pltpu.VMEM((2 * I, H), jnp.bfloat16),
                        pltpu.VMEM((H, I), jnp.bfloat16)],
    )
    return pl.pallas_call(
        _gemm_body,
        grid_spec=grid_spec,
        out_shape=jax.ShapeDtypeStruct((NPAD, H), jnp.float32),
    )(eot, xs, gate_proj, up_proj, down_proj, ws)


def kernel(x, expert_indices, expert_weights, gate_proj, up_proj, down_proj):
    b, s, h = x.shape
    x2 = x.reshape(s, h)
    idx2 = expert_indices.reshape(1, s).astype(jnp.int32)
    ew = expert_weights.reshape(s)

    pos2, eot2 = _routing(idx2)
    pos = pos2.reshape(s)
    eot = eot2.reshape(NT)

    sc_scatter, sc_gather = _sc_kernels()
    xs, ws = sc_scatter(x2, pos, ew)
    ys = _gemm(eot, xs, gate_proj, up_proj, down_proj, ws)
    out = sc_gather(ys, pos)
    return out.reshape(b, s, h)
